# baseline (speedup 1.0000x reference)
import functools
import math

import jax
import jax.numpy as jnp
from jax import lax
from jax.experimental import pallas as pl
from jax.experimental.pallas import tpu as pltpu
from jax.experimental.pallas import tpu_sc as plsc

CUTOFF = 0.8

E = 6400000
EF = 1600000

NC = 2
NS = 16
NW = NC * NS
PER_W = EF // NW
CHUNK = 10000
NCHUNK = PER_W // CHUNK
L = 16


def _sc_gather_body(vecf_hbm, dist_hbm, idx_hbm, i3_hbm, ovec_hbm, odist_hbm,
                    idx_v, i3_v, vec_v, dist_v, sem_a, sem_b):
    wid = lax.axis_index("s") * NC + lax.axis_index("c")
    base_w = wid * PER_W

    for c in range(NCHUNK):
        base = base_w + c * CHUNK
        pltpu.sync_copy(idx_hbm.at[pl.ds(base, CHUNK)], idx_v)
        pltpu.sync_copy(i3_hbm.at[pl.ds(base * 3, CHUNK * 3)], i3_v)

        cp_v = pltpu.async_copy(vecf_hbm.at[i3_v], vec_v, sem_a)
        cp_d = pltpu.async_copy(dist_hbm.at[idx_v], dist_v, sem_b)
        cp_v.wait()
        cp_d.wait()
        pltpu.sync_copy(vec_v, ovec_hbm.at[pl.ds(base * 3, CHUNK * 3)])
        pltpu.sync_copy(dist_v, odist_hbm.at[pl.ds(base, CHUNK)])


_sc_gather = functools.partial(
    pl.kernel,
    mesh=plsc.VectorSubcoreMesh(core_axis_name="c", subcore_axis_name="s"),
    out_type=[
        jax.ShapeDtypeStruct((EF * 3,), jnp.float32),
        jax.ShapeDtypeStruct((EF,), jnp.float32),
    ],
    scratch_types=[
        pltpu.VMEM((CHUNK,), jnp.int32),
        pltpu.VMEM((CHUNK * 3,), jnp.int32),
        pltpu.VMEM((CHUNK * 3,), jnp.float32),
        pltpu.VMEM((CHUNK,), jnp.float32),
        pltpu.SemaphoreType.DMA,
        pltpu.SemaphoreType.DMA,
    ],
)(_sc_gather_body)


def _tc_switch_body(d_ref, sw_ref, m_ref):
    d = d_ref[...]
    x = d * (math.pi / CUTOFF)
    s = 0.5 * (jnp.cos(x) + 1.0)
    m = d < CUTOFF
    sw_ref[...] = jnp.where(m, s, 0.0)
    m_ref[...] = m


def _tc_switch(dist_f):
    d2 = dist_f.reshape(12500, 128)
    sw, m = pl.pallas_call(
        _tc_switch_body,
        out_shape=[
            jax.ShapeDtypeStruct((12500, 128), jnp.float32),
            jax.ShapeDtypeStruct((12500, 128), jnp.bool_),
        ],
    )(d2)
    return sw.reshape(EF), m.reshape(EF)


def kernel(vec, distances, filter_indices):
    vec_flat = vec.reshape(E * 3)
    i3 = (filter_indices[:, None] * 3 + jnp.arange(3, dtype=jnp.int32)).reshape(EF * 3)
    vecf_flat, dist_f = _sc_gather(vec_flat, distances, filter_indices, i3)
    switch, mask = _tc_switch(dist_f)
    return vecf_flat.reshape(EF, 3), dist_f, switch, mask


# speedup vs baseline: 19.1479x; 19.1479x over previous
import functools
import math

import jax
import jax.numpy as jnp
from jax import lax
from jax.experimental import pallas as pl
from jax.experimental.pallas import tpu as pltpu
from jax.experimental.pallas import tpu_sc as plsc

CUTOFF = 0.8

E = 6400000
EF = 1600000

NC = 2
NS = 16
NW = NC * NS
PER_W = EF // NW
CHUNK = 10000
NCHUNK = PER_W // CHUNK


def _sc_gather_body(vx_hbm, vy_hbm, vz_hbm, dist_hbm, idx_hbm,
                    ox_hbm, oy_hbm, oz_hbm, od_hbm,
                    idx_v, x_v, y_v, z_v, d_v, s0, s1, s2, s3):
    wid = lax.axis_index("s") * NC + lax.axis_index("c")
    base_w = wid * PER_W
    for c in range(NCHUNK):
        base = base_w + c * CHUNK
        pltpu.sync_copy(idx_hbm.at[pl.ds(base, CHUNK)], idx_v)
        cps = [
            pltpu.async_copy(vx_hbm.at[idx_v], x_v, s0),
            pltpu.async_copy(vy_hbm.at[idx_v], y_v, s1),
            pltpu.async_copy(vz_hbm.at[idx_v], z_v, s2),
            pltpu.async_copy(dist_hbm.at[idx_v], d_v, s3),
        ]
        for cp in cps:
            cp.wait()
        pltpu.sync_copy(x_v, ox_hbm.at[pl.ds(base, CHUNK)])
        pltpu.sync_copy(y_v, oy_hbm.at[pl.ds(base, CHUNK)])
        pltpu.sync_copy(z_v, oz_hbm.at[pl.ds(base, CHUNK)])
        pltpu.sync_copy(d_v, od_hbm.at[pl.ds(base, CHUNK)])


_sc_gather = functools.partial(
    pl.kernel,
    mesh=plsc.VectorSubcoreMesh(core_axis_name="c", subcore_axis_name="s"),
    out_type=[jax.ShapeDtypeStruct((EF,), jnp.float32)] * 4,
    scratch_types=[
        pltpu.VMEM((CHUNK,), jnp.int32),
        pltpu.VMEM((CHUNK,), jnp.float32),
        pltpu.VMEM((CHUNK,), jnp.float32),
        pltpu.VMEM((CHUNK,), jnp.float32),
        pltpu.VMEM((CHUNK,), jnp.float32),
        pltpu.SemaphoreType.DMA,
        pltpu.SemaphoreType.DMA,
        pltpu.SemaphoreType.DMA,
        pltpu.SemaphoreType.DMA,
    ],
)(_sc_gather_body)


def _tc_switch_body(d_ref, sw_ref, m_ref):
    d = d_ref[...]
    x = d * (math.pi / CUTOFF)
    s = 0.5 * (jnp.cos(x) + 1.0)
    m = d < CUTOFF
    sw_ref[...] = jnp.where(m, s, 0.0)
    m_ref[...] = m


def _tc_switch(dist_f):
    d2 = dist_f.reshape(12500, 128)
    sw, m = pl.pallas_call(
        _tc_switch_body,
        out_shape=[
            jax.ShapeDtypeStruct((12500, 128), jnp.float32),
            jax.ShapeDtypeStruct((12500, 128), jnp.bool_),
        ],
    )(d2)
    return sw.reshape(EF), m.reshape(EF)


def kernel(vec, distances, filter_indices):
    vx, vy, vz = vec[:, 0], vec[:, 1], vec[:, 2]
    xf, yf, zf, dist_f = _sc_gather(vx, vy, vz, distances, filter_indices)
    vec_f = jnp.stack([xf, yf, zf], axis=1)
    switch, mask = _tc_switch(dist_f)
    return vec_f, dist_f, switch, mask


# speedup vs baseline: 21.7706x; 1.1370x over previous
import functools
import math

import jax
import jax.numpy as jnp
from jax import lax
from jax.experimental import pallas as pl
from jax.experimental.pallas import tpu as pltpu
from jax.experimental.pallas import tpu_sc as plsc

CUTOFF = 0.8

E = 6400000
EF = 1600000

NC = 2
NS = 16
NW = NC * NS
PER_W = EF // NW

DCHUNK = 10000
NDCHUNK = PER_W // DCHUNK
VCHUNK = 10000
NVCHUNK = PER_W // VCHUNK


def _sc_dist_body(dist_hbm, idx_hbm, od_hbm, idx_v0, idx_v1, d_v0, d_v1,
                  sg, sw):
    idx_v = [idx_v0, idx_v1]
    d_v = [d_v0, d_v1]
    wid = lax.axis_index("s") * NC + lax.axis_index("c")
    base_w = wid * PER_W
    cp_g = [None, None]
    cp_w = [None, None]
    for c in range(NDCHUNK):
        b = c & 1
        if cp_w[b] is not None:
            cp_w[b].wait()
        pltpu.sync_copy(idx_hbm.at[pl.ds(base_w + c * DCHUNK, DCHUNK)],
                        idx_v[b])
        cp_g[b] = pltpu.async_copy(dist_hbm.at[idx_v[b]], d_v[b], sg)
        if c > 0:
            pb = 1 - b
            cp_g[pb].wait()
            cp_w[pb] = pltpu.async_copy(
                d_v[pb],
                od_hbm.at[pl.ds(base_w + (c - 1) * DCHUNK, DCHUNK)], sw)
    lb = (NDCHUNK - 1) & 1
    cp_g[lb].wait()
    pltpu.sync_copy(d_v[lb],
                    od_hbm.at[pl.ds(base_w + (NDCHUNK - 1) * DCHUNK, DCHUNK)])
    if NDCHUNK > 1:
        cp_w[1 - lb].wait()


_sc_dist = functools.partial(
    pl.kernel,
    mesh=plsc.VectorSubcoreMesh(core_axis_name="c", subcore_axis_name="s"),
    out_type=[jax.ShapeDtypeStruct((EF,), jnp.float32)],
    scratch_types=[
        pltpu.VMEM((DCHUNK,), jnp.int32),
        pltpu.VMEM((DCHUNK,), jnp.int32),
        pltpu.VMEM((DCHUNK,), jnp.float32),
        pltpu.VMEM((DCHUNK,), jnp.float32),
        pltpu.SemaphoreType.DMA,
        pltpu.SemaphoreType.DMA,
    ],
)(_sc_dist_body)


def _sc_vec_body(vx_hbm, vy_hbm, vz_hbm, idx_hbm, ox_hbm, oy_hbm, oz_hbm,
                 idx_v0, idx_v1, x_v0, x_v1, y_v0, y_v1, z_v0, z_v1, sg, sw):
    idx_v = [idx_v0, idx_v1]
    x_v = [x_v0, x_v1]
    y_v = [y_v0, y_v1]
    z_v = [z_v0, z_v1]
    wid = lax.axis_index("s") * NC + lax.axis_index("c")
    base_w = wid * PER_W
    cp_g = [None, None]
    cp_w = [None, None]
    for c in range(NVCHUNK):
        b = c & 1
        if cp_w[b] is not None:
            for cp in cp_w[b]:
                cp.wait()
        pltpu.sync_copy(idx_hbm.at[pl.ds(base_w + c * VCHUNK, VCHUNK)],
                        idx_v[b])
        cp_g[b] = [
            pltpu.async_copy(vx_hbm.at[idx_v[b]], x_v[b], sg),
            pltpu.async_copy(vy_hbm.at[idx_v[b]], y_v[b], sg),
            pltpu.async_copy(vz_hbm.at[idx_v[b]], z_v[b], sg),
        ]
        if c > 0:
            pb = 1 - b
            pbase = base_w + (c - 1) * VCHUNK
            for cp in cp_g[pb]:
                cp.wait()
            cp_w[pb] = [
                pltpu.async_copy(x_v[pb], ox_hbm.at[pl.ds(pbase, VCHUNK)], sw),
                pltpu.async_copy(y_v[pb], oy_hbm.at[pl.ds(pbase, VCHUNK)], sw),
                pltpu.async_copy(z_v[pb], oz_hbm.at[pl.ds(pbase, VCHUNK)], sw),
            ]
    lb = (NVCHUNK - 1) & 1
    lbase = base_w + (NVCHUNK - 1) * VCHUNK
    for cp in cp_g[lb]:
        cp.wait()
    pltpu.sync_copy(x_v[lb], ox_hbm.at[pl.ds(lbase, VCHUNK)])
    pltpu.sync_copy(y_v[lb], oy_hbm.at[pl.ds(lbase, VCHUNK)])
    pltpu.sync_copy(z_v[lb], oz_hbm.at[pl.ds(lbase, VCHUNK)])
    if NVCHUNK > 1:
        for cp in cp_w[1 - lb]:
            cp.wait()


_sc_vec = functools.partial(
    pl.kernel,
    mesh=plsc.VectorSubcoreMesh(core_axis_name="c", subcore_axis_name="s"),
    out_type=[jax.ShapeDtypeStruct((EF,), jnp.float32)] * 3,
    scratch_types=[
        pltpu.VMEM((VCHUNK,), jnp.int32),
        pltpu.VMEM((VCHUNK,), jnp.int32),
        pltpu.VMEM((VCHUNK,), jnp.float32),
        pltpu.VMEM((VCHUNK,), jnp.float32),
        pltpu.VMEM((VCHUNK,), jnp.float32),
        pltpu.VMEM((VCHUNK,), jnp.float32),
        pltpu.VMEM((VCHUNK,), jnp.float32),
        pltpu.VMEM((VCHUNK,), jnp.float32),
        pltpu.SemaphoreType.DMA,
        pltpu.SemaphoreType.DMA,
    ],
)(_sc_vec_body)


def _tc_switch_body(d_ref, sw_ref, m_ref):
    d = d_ref[...]
    x = d * (math.pi / CUTOFF)
    s = 0.5 * (jnp.cos(x) + 1.0)
    m = d < CUTOFF
    sw_ref[...] = jnp.where(m, s, 0.0)
    m_ref[...] = m


def _tc_switch(dist_f):
    d2 = dist_f.reshape(12500, 128)
    sw, m = pl.pallas_call(
        _tc_switch_body,
        out_shape=[
            jax.ShapeDtypeStruct((12500, 128), jnp.float32),
            jax.ShapeDtypeStruct((12500, 128), jnp.bool_),
        ],
    )(d2)
    return sw.reshape(EF), m.reshape(EF)


def kernel(vec, distances, filter_indices):
    (dist_f,) = _sc_dist(distances, filter_indices)
    vx, vy, vz = vec[:, 0], vec[:, 1], vec[:, 2]
    xf, yf, zf = _sc_vec(vx, vy, vz, filter_indices)
    vec_f = jnp.stack([xf, yf, zf], axis=1)
    switch, mask = _tc_switch(dist_f)
    return vec_f, dist_f, switch, mask
